# baseline (speedup 1.0000x reference)
import functools

import jax
import jax.numpy as jnp
from jax import lax
from jax.experimental import pallas as pl
from jax.experimental.pallas import tpu as pltpu
from jax.experimental.pallas import tpu_sc as plsc

N_NODES = 10000
N_EDGES = 320000
D = 128

NC = 2
NS = 16
NW = NC * NS
E_PER_TILE = N_EDGES // NW
CH = 80
K = 3
E_PAD = 10320
NCHUNK = E_PAD // CH
NB = NCHUNK // K
TRASH = N_NODES
ACC_ROWS = 10016
STRIPE = 624
REM = N_NODES - NS * STRIPE
ZREM = ACC_ROWS - NS * STRIPE


def _sc_scatter_add():
    mesh = plsc.VectorSubcoreMesh(
        core_axis_name="c", subcore_axis_name="s", num_cores=NC, num_subcores=NS
    )

    @functools.partial(
        pl.kernel,
        out_type=jax.ShapeDtypeStruct((NC, N_NODES, D), jnp.float32),
        mesh=mesh,
        scratch_types=[
            pltpu.VMEM_SHARED((ACC_ROWS, D), jnp.float32),
            pltpu.VMEM((NCHUNK, CH), jnp.int32),
            pltpu.VMEM((K, CH), jnp.int32),
            pltpu.VMEM((K, CH), jnp.int32),
            pltpu.VMEM((K, CH, D), jnp.float32),
            [pltpu.SemaphoreType.DMA] * K,
            [pltpu.SemaphoreType.DMA] * K,
        ],
    )
    def sc_kernel(x_hbm, packed_hbm, zeros_hbm, out_hbm,
                  acc, packedv, srcc, dstc, rows, gsems, ssems):
        c = lax.axis_index("c")
        s = lax.axis_index("s")
        wid = c * NS + s

        pltpu.sync_copy(zeros_hbm, acc.at[pl.ds(s * STRIPE, STRIPE)])

        @pl.when(s == NS - 1)
        def _():
            pltpu.sync_copy(
                zeros_hbm.at[pl.ds(0, ZREM)],
                acc.at[pl.ds(NS * STRIPE, ZREM)],
            )

        pltpu.sync_copy(packed_hbm.at[wid], packedv)

        plsc.subcore_barrier()

        def unpack(i, slot):
            for k in range(CH // 16):
                p = packedv[i, pl.ds(k * 16, 16)]
                srcc[slot, pl.ds(k * 16, 16)] = p & 0xFFFF
                dstc[slot, pl.ds(k * 16, 16)] = p >> 16

        def gather(slot):
            return pltpu.async_copy(x_hbm.at[srcc.at[slot]], rows.at[slot],
                                    gsems[slot])

        def gather_wait(slot):
            pltpu.make_async_copy(x_hbm.at[srcc.at[slot]], rows.at[slot],
                                  gsems[slot]).wait()

        def scatter(slot):
            return pltpu.async_copy(rows.at[slot], acc.at[dstc.at[slot]],
                                    ssems[slot], add=True)

        def scatter_wait(slot):
            pltpu.make_async_copy(rows.at[slot], acc.at[dstc.at[slot]],
                                  ssems[slot]).wait()

        for k in range(K):
            unpack(k, k)
            gather(k)

        def block(j, carry):
            for k in range(K):
                gather_wait(k)
                scatter(k)
            for k in range(K):
                scatter_wait(k)

                @pl.when(j < NB - 1)
                def _():
                    unpack((j + 1) * K + k, k)
                    gather(k)

            return carry

        lax.fori_loop(0, NB, block, 0)

        plsc.subcore_barrier()

        pltpu.sync_copy(
            acc.at[pl.ds(s * STRIPE, STRIPE)],
            out_hbm.at[c, pl.ds(s * STRIPE, STRIPE)],
        )

        @pl.when(s == NS - 1)
        def _():
            pltpu.sync_copy(
                acc.at[pl.ds(NS * STRIPE, REM)],
                out_hbm.at[c, pl.ds(NS * STRIPE, REM)],
            )

    return sc_kernel


def _tc_combine_matmul(partials, W, b):
    BLK = 1000

    def tc_body(p_ref, w_ref, b_ref, o_ref):
        acc = p_ref[0] + p_ref[1]
        o_ref[...] = (
            jnp.dot(acc, w_ref[...], preferred_element_type=jnp.float32)
            + b_ref[...]
        )

    return pl.pallas_call(
        tc_body,
        grid=(N_NODES // BLK,),
        in_specs=[
            pl.BlockSpec((NC, BLK, D), lambda i: (0, i, 0)),
            pl.BlockSpec((D, D), lambda i: (0, 0)),
            pl.BlockSpec((1, D), lambda i: (0, 0)),
        ],
        out_specs=pl.BlockSpec((BLK, D), lambda i: (i, 0)),
        out_shape=jax.ShapeDtypeStruct((N_NODES, D), jnp.float32),
    )(partials, W, b.reshape(1, D))


def kernel(x, edge_index, W, b):
    src = edge_index[0].astype(jnp.int32).reshape(NW, E_PER_TILE)
    dst = edge_index[1].astype(jnp.int32).reshape(NW, E_PER_TILE)
    pad = E_PAD - E_PER_TILE
    src = jnp.concatenate([src, jnp.zeros((NW, pad), jnp.int32)], axis=1)
    trash = TRASH + (jnp.arange(NW, dtype=jnp.int32) % NS)[:, None]
    dst = jnp.concatenate([dst, jnp.broadcast_to(trash, (NW, pad))], axis=1)
    packed = (src | (dst << 16)).reshape(NW, NCHUNK, CH)
    zeros = jnp.zeros((STRIPE, D), jnp.float32)
    partials = _sc_scatter_add()(x, packed, zeros)
    return _tc_combine_matmul(partials, W, b)


# speedup vs baseline: 1.9498x; 1.9498x over previous
import functools

import jax
import jax.numpy as jnp
from jax import lax
from jax.experimental import pallas as pl
from jax.experimental.pallas import tpu as pltpu
from jax.experimental.pallas import tpu_sc as plsc

N_NODES = 10000
N_EDGES = 320000
D = 128

NC = 2
NS = 16
NW = NC * NS
E_PER_TILE = N_EDGES // NW
CH = 96
NCHUNK = 105
E_PAD = NCHUNK * CH
TRASH = N_NODES
ACC_ROWS = 10016
STRIPE = 624
REM = N_NODES - NS * STRIPE
ZREM = ACC_ROWS - NS * STRIPE


def _sc_scatter_add():
    mesh = plsc.VectorSubcoreMesh(
        core_axis_name="c", subcore_axis_name="s", num_cores=NC, num_subcores=NS
    )

    @functools.partial(
        pl.kernel,
        out_type=jax.ShapeDtypeStruct((NC, N_NODES, D), jnp.float32),
        mesh=mesh,
        scratch_types=[
            pltpu.VMEM_SHARED((ACC_ROWS, D), jnp.float32),
            pltpu.VMEM((NCHUNK, CH), jnp.int32),
            pltpu.VMEM((2, CH), jnp.int32),
            pltpu.VMEM((2, CH), jnp.int32),
            pltpu.VMEM((2, CH, D), jnp.float32),
            pltpu.SemaphoreType.DMA,
            pltpu.SemaphoreType.DMA,
            pltpu.SemaphoreType.DMA,
            pltpu.SemaphoreType.DMA,
        ],
    )
    def sc_kernel(x_hbm, packed_hbm, zeros_hbm, out_hbm,
                  acc, packedv, srcc, dstc, rows, gsem0, gsem1, ssem0, ssem1):
        c = lax.axis_index("c")
        s = lax.axis_index("s")
        wid = c * NS + s

        pltpu.sync_copy(zeros_hbm, acc.at[pl.ds(s * STRIPE, STRIPE)])

        @pl.when(s == NS - 1)
        def _():
            pltpu.sync_copy(
                zeros_hbm.at[pl.ds(0, ZREM)],
                acc.at[pl.ds(NS * STRIPE, ZREM)],
            )

        pltpu.sync_copy(packed_hbm.at[wid], packedv)

        plsc.subcore_barrier()

        def unpack(i, slot):
            for k in range(CH // 16):
                p = packedv[i, pl.ds(k * 16, 16)]
                srcc[slot, pl.ds(k * 16, 16)] = p & 0xFFFF
                dstc[slot, pl.ds(k * 16, 16)] = p >> 16

        def gather(slot, sem):
            return pltpu.async_copy(x_hbm.at[srcc.at[slot]], rows.at[slot], sem)

        def gather_wait(slot, sem):
            pltpu.make_async_copy(x_hbm.at[srcc.at[slot]], rows.at[slot], sem).wait()

        def scatter(slot, sem):
            return pltpu.async_copy(rows.at[slot], acc.at[dstc.at[slot]], sem, add=True)

        def scatter_wait(slot, sem):
            pltpu.make_async_copy(rows.at[slot], acc.at[dstc.at[slot]], sem).wait()

        unpack(0, 0)
        gather(0, gsem0)

        def pair(j, carry):
            i0 = 2 * j
            gather_wait(0, gsem0)

            @pl.when(j > 0)
            def _():
                scatter_wait(1, ssem1)

            unpack(i0 + 1, 1)
            gather(1, gsem1)
            scatter(0, ssem0)

            gather_wait(1, gsem1)
            scatter_wait(0, ssem0)
            unpack(i0 + 2, 0)
            gather(0, gsem0)
            scatter(1, ssem1)
            return carry

        lax.fori_loop(0, (NCHUNK - 1) // 2, pair, 0)

        gather_wait(0, gsem0)
        scatter_wait(1, ssem1)
        scatter(0, ssem0)
        scatter_wait(0, ssem0)

        plsc.subcore_barrier()

        pltpu.sync_copy(
            acc.at[pl.ds(s * STRIPE, STRIPE)],
            out_hbm.at[c, pl.ds(s * STRIPE, STRIPE)],
        )

        @pl.when(s == NS - 1)
        def _():
            pltpu.sync_copy(
                acc.at[pl.ds(NS * STRIPE, REM)],
                out_hbm.at[c, pl.ds(NS * STRIPE, REM)],
            )

    return sc_kernel


def _tc_combine_matmul(partials, W, b):
    BLK = 1000

    def tc_body(p_ref, w_ref, b_ref, o_ref):
        acc = p_ref[0] + p_ref[1]
        o_ref[...] = (
            jnp.dot(acc, w_ref[...], preferred_element_type=jnp.float32)
            + b_ref[...]
        )

    return pl.pallas_call(
        tc_body,
        grid=(N_NODES // BLK,),
        in_specs=[
            pl.BlockSpec((NC, BLK, D), lambda i: (0, i, 0)),
            pl.BlockSpec((D, D), lambda i: (0, 0)),
            pl.BlockSpec((1, D), lambda i: (0, 0)),
        ],
        out_specs=pl.BlockSpec((BLK, D), lambda i: (i, 0)),
        out_shape=jax.ShapeDtypeStruct((N_NODES, D), jnp.float32),
    )(partials, W, b.reshape(1, D))


def kernel(x, edge_index, W, b):
    src = edge_index[0].astype(jnp.int32).reshape(NW, E_PER_TILE)
    dst = edge_index[1].astype(jnp.int32).reshape(NW, E_PER_TILE)
    pad = E_PAD - E_PER_TILE
    src = jnp.concatenate([src, jnp.zeros((NW, pad), jnp.int32)], axis=1)
    trash = TRASH + (jnp.arange(NW, dtype=jnp.int32) % NS)[:, None]
    dst = jnp.concatenate([dst, jnp.broadcast_to(trash, (NW, pad))], axis=1)
    packed = (src | (dst << 16)).reshape(NW, NCHUNK, CH)
    zeros = jnp.zeros((STRIPE, D), jnp.float32)
    partials = _sc_scatter_add()(x, packed, zeros)
    return _tc_combine_matmul(partials, W, b)


# speedup vs baseline: 2.0147x; 1.0333x over previous
import functools

import jax
import jax.numpy as jnp
from jax import lax
from jax.experimental import pallas as pl
from jax.experimental.pallas import tpu as pltpu
from jax.experimental.pallas import tpu_sc as plsc

N_NODES = 10000
N_EDGES = 320000
D = 128

NC = 2
NS = 16
NW = NC * NS
E_PER_TILE = N_EDGES // NW
CH = 64
NCHUNK = 157
E_PAD = NCHUNK * CH
TRASH = N_NODES
ACC_ROWS = 10016
STRIPE = 624
REM = N_NODES - NS * STRIPE
ZREM = ACC_ROWS - NS * STRIPE


def _sc_scatter_add():
    mesh = plsc.VectorSubcoreMesh(
        core_axis_name="c", subcore_axis_name="s", num_cores=NC, num_subcores=NS
    )

    @functools.partial(
        pl.kernel,
        out_type=jax.ShapeDtypeStruct((NC, N_NODES, D), jnp.float32),
        mesh=mesh,
        scratch_types=[
            pltpu.VMEM_SHARED((ACC_ROWS, D), jnp.float32),
            pltpu.VMEM((NCHUNK, CH), jnp.int32),
            pltpu.VMEM((2, CH), jnp.int32),
            pltpu.VMEM((2, CH), jnp.int32),
            pltpu.VMEM((2, CH, D), jnp.float32),
            pltpu.SemaphoreType.DMA,
            pltpu.SemaphoreType.DMA,
            pltpu.SemaphoreType.DMA,
            pltpu.SemaphoreType.DMA,
        ],
    )
    def sc_kernel(x_hbm, packed_hbm, zeros_hbm, out_hbm,
                  acc, packedv, srcc, dstc, rows, gsem0, gsem1, ssem0, ssem1):
        c = lax.axis_index("c")
        s = lax.axis_index("s")
        wid = c * NS + s

        pltpu.sync_copy(zeros_hbm, acc.at[pl.ds(s * STRIPE, STRIPE)])

        @pl.when(s == NS - 1)
        def _():
            pltpu.sync_copy(
                zeros_hbm.at[pl.ds(0, ZREM)],
                acc.at[pl.ds(NS * STRIPE, ZREM)],
            )

        pltpu.sync_copy(packed_hbm.at[wid], packedv)

        plsc.subcore_barrier()

        def unpack(i, slot):
            for k in range(CH // 16):
                p = packedv[i, pl.ds(k * 16, 16)]
                srcc[slot, pl.ds(k * 16, 16)] = p & 0xFFFF
                dstc[slot, pl.ds(k * 16, 16)] = p >> 16

        def gather(slot, sem):
            return pltpu.async_copy(x_hbm.at[srcc.at[slot]], rows.at[slot], sem)

        def gather_wait(slot, sem):
            pltpu.make_async_copy(x_hbm.at[srcc.at[slot]], rows.at[slot], sem).wait()

        def scatter(slot, sem):
            return pltpu.async_copy(rows.at[slot], acc.at[dstc.at[slot]], sem, add=True)

        def scatter_wait(slot, sem):
            pltpu.make_async_copy(rows.at[slot], acc.at[dstc.at[slot]], sem).wait()

        unpack(0, 0)
        gather(0, gsem0)

        def pair(j, carry):
            i0 = 2 * j
            gather_wait(0, gsem0)

            @pl.when(j > 0)
            def _():
                scatter_wait(1, ssem1)

            unpack(i0 + 1, 1)
            gather(1, gsem1)
            scatter(0, ssem0)

            gather_wait(1, gsem1)
            scatter_wait(0, ssem0)
            unpack(i0 + 2, 0)
            gather(0, gsem0)
            scatter(1, ssem1)
            return carry

        lax.fori_loop(0, (NCHUNK - 1) // 2, pair, 0)

        gather_wait(0, gsem0)
        scatter_wait(1, ssem1)
        scatter(0, ssem0)
        scatter_wait(0, ssem0)

        plsc.subcore_barrier()

        pltpu.sync_copy(
            acc.at[pl.ds(s * STRIPE, STRIPE)],
            out_hbm.at[c, pl.ds(s * STRIPE, STRIPE)],
        )

        @pl.when(s == NS - 1)
        def _():
            pltpu.sync_copy(
                acc.at[pl.ds(NS * STRIPE, REM)],
                out_hbm.at[c, pl.ds(NS * STRIPE, REM)],
            )

    return sc_kernel


def _tc_combine_matmul(partials, W, b):
    BLK = 1000

    def tc_body(p_ref, w_ref, b_ref, o_ref):
        acc = p_ref[0] + p_ref[1]
        o_ref[...] = (
            jnp.dot(acc, w_ref[...], preferred_element_type=jnp.float32)
            + b_ref[...]
        )

    return pl.pallas_call(
        tc_body,
        grid=(N_NODES // BLK,),
        in_specs=[
            pl.BlockSpec((NC, BLK, D), lambda i: (0, i, 0)),
            pl.BlockSpec((D, D), lambda i: (0, 0)),
            pl.BlockSpec((1, D), lambda i: (0, 0)),
        ],
        out_specs=pl.BlockSpec((BLK, D), lambda i: (i, 0)),
        out_shape=jax.ShapeDtypeStruct((N_NODES, D), jnp.float32),
    )(partials, W, b.reshape(1, D))


def kernel(x, edge_index, W, b):
    src = edge_index[0].astype(jnp.int32).reshape(NW, E_PER_TILE)
    dst = edge_index[1].astype(jnp.int32).reshape(NW, E_PER_TILE)
    pad = E_PAD - E_PER_TILE
    src = jnp.concatenate([src, jnp.zeros((NW, pad), jnp.int32)], axis=1)
    trash = TRASH + (jnp.arange(NW, dtype=jnp.int32) % NS)[:, None]
    dst = jnp.concatenate([dst, jnp.broadcast_to(trash, (NW, pad))], axis=1)
    packed = (src | (dst << 16)).reshape(NW, NCHUNK, CH)
    zeros = jnp.zeros((STRIPE, D), jnp.float32)
    partials = _sc_scatter_add()(x, packed, zeros)
    return _tc_combine_matmul(partials, W, b)


# speedup vs baseline: 2.6921x; 1.3362x over previous
import functools

import jax
import jax.numpy as jnp
from jax import lax
from jax.experimental import pallas as pl
from jax.experimental.pallas import tpu as pltpu
from jax.experimental.pallas import tpu_sc as plsc

N_NODES = 10000
N_EDGES = 320000
D = 128

NC = 2
NS = 16
NW = NC * NS
E_PER_TILE = N_EDGES // NW
CH = 80
NCHUNK = 125
E_PAD = NCHUNK * CH
TRASH = N_NODES
ACC_ROWS = 10016
STRIPE = 624
REM = N_NODES - NS * STRIPE
ZREM = ACC_ROWS - NS * STRIPE


def _sc_scatter_add():
    mesh = plsc.VectorSubcoreMesh(
        core_axis_name="c", subcore_axis_name="s", num_cores=NC, num_subcores=NS
    )

    @functools.partial(
        pl.kernel,
        out_type=jax.ShapeDtypeStruct((NC, N_NODES, D), jnp.float32),
        mesh=mesh,
        scratch_types=[
            pltpu.VMEM_SHARED((ACC_ROWS, D), jnp.float32),
            pltpu.VMEM((NCHUNK, CH), jnp.int32),
            pltpu.VMEM((2, CH), jnp.int32),
            pltpu.VMEM((2, CH), jnp.int32),
            pltpu.VMEM((2, CH, D), jnp.float32),
            pltpu.SemaphoreType.DMA,
            pltpu.SemaphoreType.DMA,
            pltpu.SemaphoreType.DMA,
            pltpu.SemaphoreType.DMA,
        ],
    )
    def sc_kernel(x_hbm, packed_hbm, zeros_hbm, out_hbm,
                  acc, packedv, srcc, dstc, rows, gsem0, gsem1, ssem0, ssem1):
        c = lax.axis_index("c")
        s = lax.axis_index("s")
        wid = c * NS + s

        pltpu.sync_copy(zeros_hbm, acc.at[pl.ds(s * STRIPE, STRIPE)])

        @pl.when(s == NS - 1)
        def _():
            pltpu.sync_copy(
                zeros_hbm.at[pl.ds(0, ZREM)],
                acc.at[pl.ds(NS * STRIPE, ZREM)],
            )

        pltpu.sync_copy(packed_hbm.at[wid], packedv)

        plsc.subcore_barrier()

        def unpack(i, slot):
            for k in range(CH // 16):
                p = packedv[i, pl.ds(k * 16, 16)]
                srcc[slot, pl.ds(k * 16, 16)] = p & 0xFFFF
                dstc[slot, pl.ds(k * 16, 16)] = p >> 16

        def gather(slot, sem):
            return pltpu.async_copy(x_hbm.at[srcc.at[slot]], rows.at[slot], sem)

        def gather_wait(slot, sem):
            pltpu.make_async_copy(x_hbm.at[srcc.at[slot]], rows.at[slot], sem).wait()

        def scatter(slot, sem):
            return pltpu.async_copy(rows.at[slot], acc.at[dstc.at[slot]], sem, add=True)

        def scatter_wait(slot, sem):
            pltpu.make_async_copy(rows.at[slot], acc.at[dstc.at[slot]], sem).wait()

        unpack(0, 0)
        gather(0, gsem0)

        def pair(j, carry):
            i0 = 2 * j
            gather_wait(0, gsem0)

            @pl.when(j > 0)
            def _():
                scatter_wait(1, ssem1)

            unpack(i0 + 1, 1)
            gather(1, gsem1)
            scatter(0, ssem0)

            gather_wait(1, gsem1)
            scatter_wait(0, ssem0)
            unpack(i0 + 2, 0)
            gather(0, gsem0)
            scatter(1, ssem1)
            return carry

        lax.fori_loop(0, (NCHUNK - 1) // 2, pair, 0)

        gather_wait(0, gsem0)
        scatter_wait(1, ssem1)
        scatter(0, ssem0)
        scatter_wait(0, ssem0)

        plsc.subcore_barrier()

        pltpu.sync_copy(
            acc.at[pl.ds(s * STRIPE, STRIPE)],
            out_hbm.at[c, pl.ds(s * STRIPE, STRIPE)],
        )

        @pl.when(s == NS - 1)
        def _():
            pltpu.sync_copy(
                acc.at[pl.ds(NS * STRIPE, REM)],
                out_hbm.at[c, pl.ds(NS * STRIPE, REM)],
            )

    return sc_kernel


def _tc_combine_matmul(partials, W, b):
    BLK = 1000

    def tc_body(p_ref, w_ref, b_ref, o_ref):
        acc = p_ref[0] + p_ref[1]
        o_ref[...] = (
            jnp.dot(acc, w_ref[...], preferred_element_type=jnp.float32)
            + b_ref[...]
        )

    return pl.pallas_call(
        tc_body,
        grid=(N_NODES // BLK,),
        in_specs=[
            pl.BlockSpec((NC, BLK, D), lambda i: (0, i, 0)),
            pl.BlockSpec((D, D), lambda i: (0, 0)),
            pl.BlockSpec((1, D), lambda i: (0, 0)),
        ],
        out_specs=pl.BlockSpec((BLK, D), lambda i: (i, 0)),
        out_shape=jax.ShapeDtypeStruct((N_NODES, D), jnp.float32),
    )(partials, W, b.reshape(1, D))


def kernel(x, edge_index, W, b):
    src = edge_index[0].astype(jnp.int32).reshape(NW, E_PER_TILE)
    dst = edge_index[1].astype(jnp.int32).reshape(NW, E_PER_TILE)
    pad = E_PAD - E_PER_TILE
    src = jnp.concatenate([src, jnp.zeros((NW, pad), jnp.int32)], axis=1)
    trash = TRASH + (jnp.arange(NW, dtype=jnp.int32) % NS)[:, None]
    dst = jnp.concatenate([dst, jnp.broadcast_to(trash, (NW, pad))], axis=1)
    packed = (src | (dst << 16)).reshape(NW, NCHUNK, CH)
    zeros = jnp.zeros((STRIPE, D), jnp.float32)
    partials = _sc_scatter_add()(x, packed, zeros)
    return _tc_combine_matmul(partials, W, b)


# speedup vs baseline: 2.6944x; 1.0009x over previous
import functools

import jax
import jax.numpy as jnp
from jax import lax
from jax.experimental import pallas as pl
from jax.experimental.pallas import tpu as pltpu
from jax.experimental.pallas import tpu_sc as plsc

N_NODES = 10000
N_EDGES = 320000
D = 128

NC = 2
NS = 16
NW = NC * NS
E_PER_TILE = N_EDGES // NW
CH = 80
NCHUNK = 125
E_PAD = NCHUNK * CH
TRASH = N_NODES
ACC_ROWS = 10016
STRIPE = 624
REM = N_NODES - NS * STRIPE
ZREM = ACC_ROWS - NS * STRIPE


def _sc_scatter_add():
    mesh = plsc.VectorSubcoreMesh(
        core_axis_name="c", subcore_axis_name="s", num_cores=NC, num_subcores=NS
    )

    @functools.partial(
        pl.kernel,
        out_type=jax.ShapeDtypeStruct((NC, N_NODES, D), jnp.float32),
        mesh=mesh,
        scratch_types=[
            pltpu.VMEM_SHARED((ACC_ROWS, D), jnp.float32),
            pltpu.VMEM((NCHUNK, CH), jnp.int32),
            pltpu.VMEM((2, CH), jnp.int32),
            pltpu.VMEM((2, CH), jnp.int32),
            pltpu.VMEM((2, CH, D), jnp.float32),
            pltpu.SemaphoreType.DMA,
            pltpu.SemaphoreType.DMA,
            pltpu.SemaphoreType.DMA,
            pltpu.SemaphoreType.DMA,
        ],
    )
    def sc_kernel(x_hbm, packed_hbm, zeros_hbm, out_hbm,
                  acc, packedv, srcc, dstc, rows, gsem0, gsem1, ssem0, ssem1):
        c = lax.axis_index("c")
        s = lax.axis_index("s")
        wid = c * NS + s

        pltpu.sync_copy(zeros_hbm, acc.at[pl.ds(s * STRIPE, STRIPE)])

        @pl.when(s == NS - 1)
        def _():
            pltpu.sync_copy(
                zeros_hbm.at[pl.ds(0, ZREM)],
                acc.at[pl.ds(NS * STRIPE, ZREM)],
            )

        pltpu.sync_copy(packed_hbm.at[wid], packedv)

        plsc.subcore_barrier()

        def unpack(i, slot):
            for k in range(CH // 16):
                p = packedv[i, pl.ds(k * 16, 16)]
                srcc[slot, pl.ds(k * 16, 16)] = p & 0xFFFF
                dstc[slot, pl.ds(k * 16, 16)] = p >> 16

        def gather(slot, sem):
            return pltpu.async_copy(x_hbm.at[srcc.at[slot]], rows.at[slot], sem)

        def gather_wait(slot, sem):
            pltpu.make_async_copy(x_hbm.at[srcc.at[slot]], rows.at[slot], sem).wait()

        def scatter(slot, sem):
            return pltpu.async_copy(rows.at[slot], acc.at[dstc.at[slot]], sem, add=True)

        def scatter_wait(slot, sem):
            pltpu.make_async_copy(rows.at[slot], acc.at[dstc.at[slot]], sem).wait()

        unpack(0, 0)
        gather(0, gsem0)

        def pair(j, carry):
            i0 = 2 * j
            gather_wait(0, gsem0)

            @pl.when(j > 0)
            def _():
                scatter_wait(1, ssem1)

            scatter(0, ssem0)
            unpack(i0 + 1, 1)
            gather(1, gsem1)

            gather_wait(1, gsem1)
            scatter_wait(0, ssem0)
            scatter(1, ssem1)
            unpack(i0 + 2, 0)
            gather(0, gsem0)
            return carry

        lax.fori_loop(0, (NCHUNK - 1) // 2, pair, 0)

        gather_wait(0, gsem0)
        scatter_wait(1, ssem1)
        scatter(0, ssem0)
        scatter_wait(0, ssem0)

        plsc.subcore_barrier()

        pltpu.sync_copy(
            acc.at[pl.ds(s * STRIPE, STRIPE)],
            out_hbm.at[c, pl.ds(s * STRIPE, STRIPE)],
        )

        @pl.when(s == NS - 1)
        def _():
            pltpu.sync_copy(
                acc.at[pl.ds(NS * STRIPE, REM)],
                out_hbm.at[c, pl.ds(NS * STRIPE, REM)],
            )

    return sc_kernel


def _tc_combine_matmul(partials, W, b):
    BLK = 1000

    def tc_body(p_ref, w_ref, b_ref, o_ref):
        acc = p_ref[0] + p_ref[1]
        o_ref[...] = (
            jnp.dot(acc, w_ref[...], preferred_element_type=jnp.float32)
            + b_ref[...]
        )

    return pl.pallas_call(
        tc_body,
        grid=(N_NODES // BLK,),
        in_specs=[
            pl.BlockSpec((NC, BLK, D), lambda i: (0, i, 0)),
            pl.BlockSpec((D, D), lambda i: (0, 0)),
            pl.BlockSpec((1, D), lambda i: (0, 0)),
        ],
        out_specs=pl.BlockSpec((BLK, D), lambda i: (i, 0)),
        out_shape=jax.ShapeDtypeStruct((N_NODES, D), jnp.float32),
    )(partials, W, b.reshape(1, D))


def kernel(x, edge_index, W, b):
    src = edge_index[0].astype(jnp.int32).reshape(NW, E_PER_TILE)
    dst = edge_index[1].astype(jnp.int32).reshape(NW, E_PER_TILE)
    pad = E_PAD - E_PER_TILE
    src = jnp.concatenate([src, jnp.zeros((NW, pad), jnp.int32)], axis=1)
    trash = TRASH + (jnp.arange(NW, dtype=jnp.int32) % NS)[:, None]
    dst = jnp.concatenate([dst, jnp.broadcast_to(trash, (NW, pad))], axis=1)
    packed = (src | (dst << 16)).reshape(NW, NCHUNK, CH)
    zeros = jnp.zeros((STRIPE, D), jnp.float32)
    partials = _sc_scatter_add()(x, packed, zeros)
    return _tc_combine_matmul(partials, W, b)
